# baseline (speedup 1.0000x reference)
import functools

import jax
import jax.numpy as jnp
from jax import lax
from jax.experimental import pallas as pl
from jax.experimental.pallas import tpu as pltpu
from jax.experimental.pallas import tpu_sc as plsc

NUM_RAYS = 8192
TOTAL_SAMPLES = 524288
NC = 2
NS = 16
NW = NC * NS
RPW = NUM_RAYS // NW
L = 16
CH = 2048
VC = CH // L

_f32 = jnp.float32
_i32 = jnp.int32

_GDN = lax.GatherDimensionNumbers(
    offset_dims=(), collapsed_slice_dims=(0,), start_index_map=(0,))


def _vgather(vec, idx):
    return lax.gather(vec, idx[:, None], _GDN, slice_sizes=(1,),
                      mode=lax.GatherScatterMode.PROMISE_IN_BOUNDS)


def _rsqrt(x):
    i = lax.bitcast_convert_type(x, _i32)
    y = lax.bitcast_convert_type(jnp.int32(0x5F3759DF) - (i >> 1), _f32)
    for _ in range(3):
        y = y * (1.5 - 0.5 * x * y * y)
    return y


def _body(a_hbm, t_hbm, r0_hbm, r1_hbm, r2_hbm, n0_hbm, n1_hbm, n2_hbm,
          seg_hbm, bnd_hbm,
          mask_out, depth_out, rgb_out, nrm_out,
          bnd_v, a_buf, t_buf, s_buf,
          r0_buf, r1_buf, r2_buf, n0_buf, n1_buf, n2_buf,
          m_acc, d_acc, c_acc, n_acc, dsem):
    w = lax.axis_index("s") * NC + lax.axis_index("c")
    ray_base = pl.multiple_of(w * RPW, RPW)
    iota = lax.iota(_i32, L)

    pltpu.sync_copy(bnd_hbm, bnd_v)
    lo = jnp.min(plsc.load_gather(bnd_v, [jnp.full((L,), w, _i32)]))
    hi = jnp.min(plsc.load_gather(bnd_v, [jnp.full((L,), w + 1, _i32)]))

    zf = jnp.zeros((L,), _f32)
    for i in range(RPW // L):
        m_acc[pl.ds(i * L, L)] = zf
        d_acc[pl.ds(i * L, L)] = zf
    for i in range(3 * RPW // L):
        c_acc[pl.ds(i * L, L)] = zf
        n_acc[pl.ds(i * L, L)] = zf

    lo16 = lo & ~15
    nv = (hi - lo16 + 15) >> 4
    nk = (nv + VC - 1) // VC

    def shr(vec, d, fill):
        g = _vgather(vec, jnp.maximum(iota - d, 0))
        return jnp.where(iota < d, fill, g)

    pairs = ((a_hbm, a_buf), (t_hbm, t_buf), (seg_hbm, s_buf),
             (r0_hbm, r0_buf), (r1_hbm, r1_buf), (r2_hbm, r2_buf),
             (n0_hbm, n0_buf), (n1_hbm, n1_buf), (n2_hbm, n2_buf))

    def chunk_start(k):
        return pl.multiple_of(jnp.minimum(lo16 + k * CH, TOTAL_SAMPLES - CH), 16)

    def fire(k):
        st = chunk_start(k)
        boff = pl.multiple_of((k & 1) * CH, 16)
        for src, dst in pairs:
            pltpu.async_copy(src.at[pl.ds(st, CH)], dst.at[pl.ds(boff, CH)], dsem)

    fire(0)

    def chunk_body(k, carry):
        st = chunk_start(k)
        boff = pl.multiple_of((k & 1) * CH, 16)
        for src, dst in pairs:
            pltpu.make_async_copy(src.at[pl.ds(st, CH)],
                                  dst.at[pl.ds(boff, CH)], dsem).wait()

        @pl.when(k + 1 < nk)
        def _():
            fire(k + 1)

        start = st - boff
        nv_k = jnp.minimum(nv - k * VC, VC)

        def vec_body(v, carry2):
            carry_seg, carry_T = carry2
            g0 = lo16 + k * CH + v * L
            off = g0 - start
            gidx = g0 + iota
            a = a_buf[pl.ds(off, L)]
            tt = t_buf[pl.ds(off, L)]
            s = s_buf[pl.ds(off, L)]
            r0 = r0_buf[pl.ds(off, L)]
            r1 = r1_buf[pl.ds(off, L)]
            r2 = r2_buf[pl.ds(off, L)]
            n0 = n0_buf[pl.ds(off, L)]
            n1 = n1_buf[pl.ds(off, L)]
            n2 = n2_buf[pl.ds(off, L)]

            valid = (gidx >= lo) & (gidx < hi)
            a = jnp.where(valid, a, 0.0)
            s = jnp.where(valid, s, -2)

            q = jnp.clip(1.0 - a, 1e-10, 1.0)
            P = q
            for d in (1, 2, 4, 8):
                Pd = shr(P, d, 1.0)
                sd = shr(s, d, -3)
                P = jnp.where(s == sd, P * Pd, P)
            T_incl = jnp.where(s == carry_seg, P * carry_T, P)
            T_shift = shr(T_incl, 1, carry_T)
            s_shift = shr(s, 1, carry_seg)
            T_excl = jnp.where(s_shift == s, T_shift, 1.0)
            vw = a * T_excl

            c0 = jnp.clip(n0, -1.0, 1.0)
            c1 = jnp.clip(n1, -1.0, 1.0)
            c2 = jnp.clip(n2, -1.0, 1.0)
            rs = _rsqrt(jnp.maximum(c0 * c0 + c1 * c1 + c2 * c2, 1e-24))

            sl = jnp.clip(s - ray_base, 0, RPW - 1)
            mvalid = (s >= ray_base) & (s < ray_base + RPW)
            sl3 = sl * 3

            plsc.addupdate_scatter(m_acc, [sl], vw, mask=mvalid)
            plsc.addupdate_scatter(d_acc, [sl], vw * tt, mask=mvalid)
            plsc.addupdate_scatter(c_acc, [sl3], vw * r0, mask=mvalid)
            plsc.addupdate_scatter(c_acc, [sl3 + 1], vw * r1, mask=mvalid)
            plsc.addupdate_scatter(c_acc, [sl3 + 2], vw * r2, mask=mvalid)
            vwrs = vw * rs
            plsc.addupdate_scatter(n_acc, [sl3], vwrs * c0, mask=mvalid)
            plsc.addupdate_scatter(n_acc, [sl3 + 1], vwrs * c1, mask=mvalid)
            plsc.addupdate_scatter(n_acc, [sl3 + 2], vwrs * c2, mask=mvalid)

            new_cs = _vgather(s, jnp.full((L,), L - 1, _i32))
            new_cT = _vgather(T_incl, jnp.full((L,), L - 1, _i32))
            return new_cs, new_cT

        return lax.fori_loop(0, nv_k, vec_body, carry)

    lax.fori_loop(0, nk, chunk_body,
                  (jnp.full((L,), -1, _i32), jnp.ones((L,), _f32)))

    for i in range(RPW // L):
        mv = m_acc[pl.ds(i * L, L)]
        dv = d_acc[pl.ds(i * L, L)]
        d_acc[pl.ds(i * L, L)] = dv / (mv + 1e-10)

    pltpu.sync_copy(m_acc, mask_out.at[pl.ds(ray_base, RPW)])
    pltpu.sync_copy(d_acc, depth_out.at[pl.ds(ray_base, RPW)])
    pltpu.sync_copy(c_acc, rgb_out.at[pl.ds(ray_base * 3, 3 * RPW)])
    pltpu.sync_copy(n_acc, nrm_out.at[pl.ds(ray_base * 3, 3 * RPW)])


@jax.jit
def kernel(opacity_alpha, t, rgb, nablas, segment_ids):
    seg32 = segment_ids.astype(_i32)
    targets = (jnp.arange(NW + 1, dtype=_i32) * RPW).astype(_i32)
    bounds = jnp.searchsorted(seg32, targets).astype(_i32)
    bounds = jnp.concatenate([bounds, jnp.zeros((15,), _i32)])

    call = functools.partial(
        pl.kernel,
        mesh=plsc.VectorSubcoreMesh(core_axis_name="c", subcore_axis_name="s"),
        compiler_params=pltpu.CompilerParams(needs_layout_passes=False),
        out_type=[
            jax.ShapeDtypeStruct((NUM_RAYS,), _f32),
            jax.ShapeDtypeStruct((NUM_RAYS,), _f32),
            jax.ShapeDtypeStruct((3 * NUM_RAYS,), _f32),
            jax.ShapeDtypeStruct((3 * NUM_RAYS,), _f32),
        ],
        scratch_types=[
            pltpu.VMEM((48,), _i32),
            pltpu.VMEM((2 * CH,), _f32),
            pltpu.VMEM((2 * CH,), _f32),
            pltpu.VMEM((2 * CH,), _i32),
            pltpu.VMEM((2 * CH,), _f32),
            pltpu.VMEM((2 * CH,), _f32),
            pltpu.VMEM((2 * CH,), _f32),
            pltpu.VMEM((2 * CH,), _f32),
            pltpu.VMEM((2 * CH,), _f32),
            pltpu.VMEM((2 * CH,), _f32),
            pltpu.VMEM((RPW,), _f32),
            pltpu.VMEM((RPW,), _f32),
            pltpu.VMEM((3 * RPW,), _f32),
            pltpu.VMEM((3 * RPW,), _f32),
            pltpu.SemaphoreType.DMA,
        ],
    )(_body)
    mask, depth, rgbf, nrmf = call(
        opacity_alpha, t, rgb[:, 0], rgb[:, 1], rgb[:, 2],
        nablas[:, 0], nablas[:, 1], nablas[:, 2], seg32, bounds)
    return (mask, depth, rgbf.reshape(NUM_RAYS, 3), nrmf.reshape(NUM_RAYS, 3))


# speedup vs baseline: 1.6990x; 1.6990x over previous
import functools

import jax
import jax.numpy as jnp
from jax import lax
from jax.experimental import pallas as pl
from jax.experimental.pallas import tpu as pltpu
from jax.experimental.pallas import tpu_sc as plsc

NUM_RAYS = 8192
TOTAL_SAMPLES = 524288
NC = 2
NS = 16
NW = NC * NS
RPW = NUM_RAYS // NW
L = 16
CH = 2048
VC = CH // L

_f32 = jnp.float32
_i32 = jnp.int32

_GDN = lax.GatherDimensionNumbers(
    offset_dims=(), collapsed_slice_dims=(0,), start_index_map=(0,))


def _vgather(vec, idx):
    return lax.gather(vec, idx[:, None], _GDN, slice_sizes=(1,),
                      mode=lax.GatherScatterMode.PROMISE_IN_BOUNDS)


def _rsqrt(x):
    i = lax.bitcast_convert_type(x, _i32)
    y = lax.bitcast_convert_type(jnp.int32(0x5F3759DF) - (i >> 1), _f32)
    for _ in range(3):
        y = y * (1.5 - 0.5 * x * y * y)
    return y


def _body(a_hbm, t_hbm, r0_hbm, r1_hbm, r2_hbm, n0_hbm, n1_hbm, n2_hbm,
          seg_hbm, bnd_hbm,
          mask_out, depth_out, rgb_out, nrm_out,
          bnd_v, a_buf, t_buf, s_buf,
          r0_buf, r1_buf, r2_buf, n0_buf, n1_buf, n2_buf,
          m_acc, d_acc, c_acc, n_acc, dsem):
    w = lax.axis_index("s") * NC + lax.axis_index("c")
    ray_base = pl.multiple_of(w * RPW, RPW)
    iota = lax.iota(_i32, L)

    pltpu.sync_copy(bnd_hbm, bnd_v)
    lo = jnp.min(plsc.load_gather(bnd_v, [jnp.full((L,), w, _i32)]))
    hi = jnp.min(plsc.load_gather(bnd_v, [jnp.full((L,), w + 1, _i32)]))

    zf = jnp.zeros((L,), _f32)
    for i in range(RPW // L):
        m_acc[pl.ds(i * L, L)] = zf
        d_acc[pl.ds(i * L, L)] = zf
    for i in range(3 * RPW // L):
        c_acc[pl.ds(i * L, L)] = zf
        n_acc[pl.ds(i * L, L)] = zf

    lo16 = lo & ~15
    nv = (hi - lo16 + 15) >> 4
    nk = (nv + VC - 1) // VC

    def shr(vec, d, fill):
        g = _vgather(vec, jnp.maximum(iota - d, 0))
        return jnp.where(iota < d, fill, g)

    pairs = ((a_hbm, a_buf), (t_hbm, t_buf), (seg_hbm, s_buf),
             (r0_hbm, r0_buf), (r1_hbm, r1_buf), (r2_hbm, r2_buf),
             (n0_hbm, n0_buf), (n1_hbm, n1_buf), (n2_hbm, n2_buf))

    def chunk_start(k):
        return pl.multiple_of(jnp.minimum(lo16 + k * CH, TOTAL_SAMPLES - CH), 16)

    def fire(k):
        st = chunk_start(k)
        boff = pl.multiple_of((k & 1) * CH, 16)
        for src, dst in pairs:
            pltpu.async_copy(src.at[pl.ds(st, CH)], dst.at[pl.ds(boff, CH)], dsem)

    fire(0)

    def chunk_body(k, carry):
        st = chunk_start(k)
        boff = pl.multiple_of((k & 1) * CH, 16)
        for src, dst in pairs:
            pltpu.make_async_copy(src.at[pl.ds(st, CH)],
                                  dst.at[pl.ds(boff, CH)], dsem).wait()

        @pl.when(k + 1 < nk)
        def _():
            fire(k + 1)

        start = st - boff
        nv_k = jnp.minimum(nv - k * VC, VC)

        def vec_body(v, carry2):
            carry_seg, carry_T = carry2
            g0 = lo16 + k * CH + v * L
            off = g0 - start
            gidx = g0 + iota
            a = a_buf[pl.ds(off, L)]
            tt = t_buf[pl.ds(off, L)]
            s = s_buf[pl.ds(off, L)]
            r0 = r0_buf[pl.ds(off, L)]
            r1 = r1_buf[pl.ds(off, L)]
            r2 = r2_buf[pl.ds(off, L)]
            n0 = n0_buf[pl.ds(off, L)]
            n1 = n1_buf[pl.ds(off, L)]
            n2 = n2_buf[pl.ds(off, L)]

            valid = (gidx >= lo) & (gidx < hi)
            a = jnp.where(valid, a, 0.0)
            s = jnp.where(valid, s, -2)

            q = jnp.clip(1.0 - a, 1e-10, 1.0)
            P = q
            for d in (1, 2, 4, 8):
                Pd = shr(P, d, 1.0)
                sd = shr(s, d, -3)
                P = jnp.where(s == sd, P * Pd, P)
            T_incl = jnp.where(s == carry_seg, P * carry_T, P)
            T_shift = shr(T_incl, 1, carry_T)
            s_shift = shr(s, 1, carry_seg)
            T_excl = jnp.where(s_shift == s, T_shift, 1.0)
            vw = a * T_excl

            c0 = jnp.clip(n0, -1.0, 1.0)
            c1 = jnp.clip(n1, -1.0, 1.0)
            c2 = jnp.clip(n2, -1.0, 1.0)
            rs = _rsqrt(jnp.maximum(c0 * c0 + c1 * c1 + c2 * c2, 1e-24))

            next_s = _vgather(s, jnp.minimum(iota + 1, L - 1))
            next_s = jnp.where(iota == L - 1, -9, next_s)
            run_end = s != next_s
            is_start = s != s_shift
            rstart = plsc.cummax(jnp.where(is_start, iota, 0))
            cb_idx = jnp.maximum(rstart - 1, 0)
            has_prev = rstart > 0
            sl = jnp.clip(s - ray_base, 0, RPW - 1)
            mvalid = run_end & (s >= ray_base) & (s < ray_base + RPW)
            sl3 = sl * 3

            def seg_total(x):
                c = plsc.cumsum(x)
                cb = jnp.where(has_prev, _vgather(c, cb_idx), 0.0)
                return c - cb

            plsc.addupdate_scatter(m_acc, [sl], seg_total(vw), mask=mvalid)
            plsc.addupdate_scatter(d_acc, [sl], seg_total(vw * tt), mask=mvalid)
            plsc.addupdate_scatter(c_acc, [sl3], seg_total(vw * r0), mask=mvalid)
            plsc.addupdate_scatter(c_acc, [sl3 + 1], seg_total(vw * r1), mask=mvalid)
            plsc.addupdate_scatter(c_acc, [sl3 + 2], seg_total(vw * r2), mask=mvalid)
            vwrs = vw * rs
            plsc.addupdate_scatter(n_acc, [sl3], seg_total(vwrs * c0), mask=mvalid)
            plsc.addupdate_scatter(n_acc, [sl3 + 1], seg_total(vwrs * c1), mask=mvalid)
            plsc.addupdate_scatter(n_acc, [sl3 + 2], seg_total(vwrs * c2), mask=mvalid)

            new_cs = _vgather(s, jnp.full((L,), L - 1, _i32))
            new_cT = _vgather(T_incl, jnp.full((L,), L - 1, _i32))
            return new_cs, new_cT

        return lax.fori_loop(0, nv_k, vec_body, carry)

    lax.fori_loop(0, nk, chunk_body,
                  (jnp.full((L,), -1, _i32), jnp.ones((L,), _f32)))

    for i in range(RPW // L):
        mv = m_acc[pl.ds(i * L, L)]
        dv = d_acc[pl.ds(i * L, L)]
        d_acc[pl.ds(i * L, L)] = dv / (mv + 1e-10)

    pltpu.sync_copy(m_acc, mask_out.at[pl.ds(ray_base, RPW)])
    pltpu.sync_copy(d_acc, depth_out.at[pl.ds(ray_base, RPW)])
    pltpu.sync_copy(c_acc, rgb_out.at[pl.ds(ray_base * 3, 3 * RPW)])
    pltpu.sync_copy(n_acc, nrm_out.at[pl.ds(ray_base * 3, 3 * RPW)])


@jax.jit
def kernel(opacity_alpha, t, rgb, nablas, segment_ids):
    seg32 = segment_ids.astype(_i32)
    targets = (jnp.arange(NW + 1, dtype=_i32) * RPW).astype(_i32)
    bounds = jnp.searchsorted(seg32, targets).astype(_i32)
    bounds = jnp.concatenate([bounds, jnp.zeros((15,), _i32)])

    call = functools.partial(
        pl.kernel,
        mesh=plsc.VectorSubcoreMesh(core_axis_name="c", subcore_axis_name="s"),
        compiler_params=pltpu.CompilerParams(needs_layout_passes=False),
        out_type=[
            jax.ShapeDtypeStruct((NUM_RAYS,), _f32),
            jax.ShapeDtypeStruct((NUM_RAYS,), _f32),
            jax.ShapeDtypeStruct((3 * NUM_RAYS,), _f32),
            jax.ShapeDtypeStruct((3 * NUM_RAYS,), _f32),
        ],
        scratch_types=[
            pltpu.VMEM((48,), _i32),
            pltpu.VMEM((2 * CH,), _f32),
            pltpu.VMEM((2 * CH,), _f32),
            pltpu.VMEM((2 * CH,), _i32),
            pltpu.VMEM((2 * CH,), _f32),
            pltpu.VMEM((2 * CH,), _f32),
            pltpu.VMEM((2 * CH,), _f32),
            pltpu.VMEM((2 * CH,), _f32),
            pltpu.VMEM((2 * CH,), _f32),
            pltpu.VMEM((2 * CH,), _f32),
            pltpu.VMEM((RPW,), _f32),
            pltpu.VMEM((RPW,), _f32),
            pltpu.VMEM((3 * RPW,), _f32),
            pltpu.VMEM((3 * RPW,), _f32),
            pltpu.SemaphoreType.DMA,
        ],
    )(_body)
    mask, depth, rgbf, nrmf = call(
        opacity_alpha, t, rgb[:, 0], rgb[:, 1], rgb[:, 2],
        nablas[:, 0], nablas[:, 1], nablas[:, 2], seg32, bounds)
    return (mask, depth, rgbf.reshape(NUM_RAYS, 3), nrmf.reshape(NUM_RAYS, 3))


# speedup vs baseline: 2.3558x; 1.3865x over previous
import functools

import jax
import jax.numpy as jnp
from jax import lax
from jax.experimental import pallas as pl
from jax.experimental.pallas import tpu as pltpu
from jax.experimental.pallas import tpu_sc as plsc

NUM_RAYS = 8192
TOTAL_SAMPLES = 524288
NC = 2
NS = 16
NW = NC * NS
RPW = NUM_RAYS // NW
L = 16
CH = 2048
VC = CH // L

_f32 = jnp.float32
_i32 = jnp.int32

_GDN = lax.GatherDimensionNumbers(
    offset_dims=(), collapsed_slice_dims=(0,), start_index_map=(0,))


def _vgather(vec, idx):
    return lax.gather(vec, idx[:, None], _GDN, slice_sizes=(1,),
                      mode=lax.GatherScatterMode.PROMISE_IN_BOUNDS)


def _rsqrt(x):
    i = lax.bitcast_convert_type(x, _i32)
    y = lax.bitcast_convert_type(jnp.int32(0x5F3759DF) - (i >> 1), _f32)
    for _ in range(3):
        y = y * (1.5 - 0.5 * x * y * y)
    return y


def _body(a_hbm, t_hbm, r0_hbm, r1_hbm, r2_hbm, n0_hbm, n1_hbm, n2_hbm,
          seg_hbm, bnd_hbm,
          mask_out, depth_out, rgb_out, nrm_out,
          bnd_v, a_buf, t_buf, s_buf,
          r0_buf, r1_buf, r2_buf, n0_buf, n1_buf, n2_buf,
          m_acc, d_acc, c_acc, n_acc, dsem):
    w = lax.axis_index("s") * NC + lax.axis_index("c")
    ray_base = pl.multiple_of(w * RPW, RPW)
    iota = lax.iota(_i32, L)
    _C0 = jnp.zeros((L,), _i32)
    _C1 = _C0 + 1
    _C2 = _C0 + 2

    pltpu.sync_copy(bnd_hbm, bnd_v)
    lo = jnp.min(plsc.load_gather(bnd_v, [jnp.full((L,), w, _i32)]))
    hi = jnp.min(plsc.load_gather(bnd_v, [jnp.full((L,), w + 1, _i32)]))

    zf = jnp.zeros((L,), _f32)
    for i in range(RPW // L):
        m_acc[pl.ds(i * L, L)] = zf
        d_acc[pl.ds(i * L, L)] = zf
    for i in range(3 * RPW // L):
        c_acc[pl.ds(i * L, L)] = zf
        n_acc[pl.ds(i * L, L)] = zf

    lo16 = lo & ~15
    nv = (hi - lo16 + 15) >> 4
    nk = (nv + VC - 1) // VC

    def shr(vec, d, fill):
        g = _vgather(vec, jnp.maximum(iota - d, 0))
        return jnp.where(iota < d, fill, g)

    def chunk_start(k):
        return pl.multiple_of(jnp.minimum(lo16 + k * CH, TOTAL_SAMPLES - CH), 16)

    def copies(k, fn):
        st = chunk_start(k)
        boff = pl.multiple_of((k & 1) * CH, 16)
        fn(a_hbm.at[pl.ds(st, CH)], a_buf.at[pl.ds(boff, CH)], dsem)
        fn(t_hbm.at[pl.ds(st, CH)], t_buf.at[pl.ds(boff, CH)], dsem)
        fn(seg_hbm.at[pl.ds(st, CH)], s_buf.at[pl.ds(boff, CH)], dsem)
        fn(r0_hbm.at[pl.ds(st, CH)], r0_buf.at[pl.ds(boff, CH)], dsem)
        fn(r1_hbm.at[pl.ds(st, CH)], r1_buf.at[pl.ds(boff, CH)], dsem)
        fn(r2_hbm.at[pl.ds(st, CH)], r2_buf.at[pl.ds(boff, CH)], dsem)
        fn(n0_hbm.at[pl.ds(st, CH)], n0_buf.at[pl.ds(boff, CH)], dsem)
        fn(n1_hbm.at[pl.ds(st, CH)], n1_buf.at[pl.ds(boff, CH)], dsem)
        fn(n2_hbm.at[pl.ds(st, CH)], n2_buf.at[pl.ds(boff, CH)], dsem)

    def fire(k):
        copies(k, pltpu.async_copy)

    def chunk_body(k, carry):
        copies(k, lambda s, d, m: pltpu.make_async_copy(s, d, m).wait())

        @pl.when(k + 1 < nk)
        def _():
            fire(k + 1)

        start = chunk_start(k) - pl.multiple_of((k & 1) * CH, 16)
        nv_k = jnp.minimum(nv - k * VC, VC)

        def vec_body(v, carry2):
            carry_seg, carry_T = carry2
            g0 = lo16 + k * CH + v * L
            off = g0 - start
            gidx = g0 + iota
            a = a_buf[pl.ds(off, L)]
            tt = t_buf[pl.ds(off, L)]
            s = s_buf[pl.ds(off, L)]
            r0 = r0_buf[pl.ds(off, L)]
            r1 = r1_buf[pl.ds(off, L)]
            r2 = r2_buf[pl.ds(off, L)]
            n0 = n0_buf[pl.ds(off, L)]
            n1 = n1_buf[pl.ds(off, L)]
            n2 = n2_buf[pl.ds(off, L)]

            valid = (gidx >= lo) & (gidx < hi)
            a = jnp.where(valid, a, 0.0)
            s = jnp.where(valid, s, -2)

            q = jnp.clip(1.0 - a, 1e-10, 1.0)
            P = q
            for d in (1, 2, 4, 8):
                Pd = shr(P, d, 1.0)
                sd = shr(s, d, -3)
                P = jnp.where(s == sd, P * Pd, P)
            T_incl = jnp.where(s == carry_seg, P * carry_T, P)
            T_shift = shr(T_incl, 1, carry_T)
            s_shift = shr(s, 1, carry_seg)
            T_excl = jnp.where(s_shift == s, T_shift, 1.0)
            vw = a * T_excl

            c0 = jnp.clip(n0, -1.0, 1.0)
            c1 = jnp.clip(n1, -1.0, 1.0)
            c2 = jnp.clip(n2, -1.0, 1.0)
            rs = _rsqrt(jnp.maximum(c0 * c0 + c1 * c1 + c2 * c2, 1e-24))

            next_s = _vgather(s, jnp.minimum(iota + 1, L - 1))
            next_s = jnp.where(iota == L - 1, -9, next_s)
            run_end = s != next_s
            is_start = s != s_shift
            rstart = plsc.cummax(jnp.where(is_start, iota, 0))
            cb_idx = jnp.maximum(rstart - 1, 0)
            has_prev = rstart > 0
            sl = jnp.clip(s - ray_base, 0, RPW - 1)
            mvalid = run_end & (s >= ray_base) & (s < ray_base + RPW)
            sl3 = sl * 3

            def seg_total(x):
                c = plsc.cumsum(x)
                cb = jnp.where(has_prev, _vgather(c, cb_idx), 0.0)
                return c - cb

            plsc.addupdate_scatter(m_acc, [sl], seg_total(vw), mask=mvalid)
            plsc.addupdate_scatter(d_acc, [sl], seg_total(vw * tt), mask=mvalid)
            plsc.addupdate_scatter(c_acc, [sl3], seg_total(vw * r0), mask=mvalid)
            plsc.addupdate_scatter(c_acc, [sl3 + 1], seg_total(vw * r1), mask=mvalid)
            plsc.addupdate_scatter(c_acc, [sl3 + 2], seg_total(vw * r2), mask=mvalid)
            vwrs = vw * rs
            plsc.addupdate_scatter(n_acc, [sl3], seg_total(vwrs * c0), mask=mvalid)
            plsc.addupdate_scatter(n_acc, [sl3 + 1], seg_total(vwrs * c1), mask=mvalid)
            plsc.addupdate_scatter(n_acc, [sl3 + 2], seg_total(vwrs * c2), mask=mvalid)

            new_cs = _vgather(s, jnp.full((L,), L - 1, _i32))
            new_cT = _vgather(T_incl, jnp.full((L,), L - 1, _i32))
            return new_cs, new_cT

        return lax.fori_loop(0, nv_k, vec_body, carry)

    fire(0)
    lax.fori_loop(0, nk, chunk_body,
                  (jnp.full((L,), -1, _i32), jnp.ones((L,), _f32)))

    for i in range(RPW // L):
        mv = m_acc[pl.ds(i * L, L)]
        dv = d_acc[pl.ds(i * L, L)]
        d_acc[pl.ds(i * L, L)] = dv / (mv + 1e-10)

    pltpu.sync_copy(m_acc, mask_out.at[pl.ds(ray_base, RPW)])
    pltpu.sync_copy(d_acc, depth_out.at[pl.ds(ray_base, RPW)])
    pltpu.sync_copy(c_acc, rgb_out.at[pl.ds(ray_base * 3, 3 * RPW)])
    pltpu.sync_copy(n_acc, nrm_out.at[pl.ds(ray_base * 3, 3 * RPW)])


@jax.jit
def kernel(opacity_alpha, t, rgb, nablas, segment_ids):
    seg32 = segment_ids.astype(_i32)
    targets = jnp.arange(NW + 1, dtype=_i32) * RPW
    bounds = jnp.sum((seg32[:, None] < targets[None, :]).astype(_i32), axis=0)
    bounds = jnp.concatenate([bounds, jnp.zeros((15,), _i32)])

    call = functools.partial(
        pl.kernel,
        mesh=plsc.VectorSubcoreMesh(core_axis_name="c", subcore_axis_name="s"),
        compiler_params=pltpu.CompilerParams(needs_layout_passes=False),
        out_type=[
            jax.ShapeDtypeStruct((NUM_RAYS,), _f32),
            jax.ShapeDtypeStruct((NUM_RAYS,), _f32),
            jax.ShapeDtypeStruct((3 * NUM_RAYS,), _f32),
            jax.ShapeDtypeStruct((3 * NUM_RAYS,), _f32),
        ],
        scratch_types=[
            pltpu.VMEM((48,), _i32),
            pltpu.VMEM((2 * CH,), _f32),
            pltpu.VMEM((2 * CH,), _f32),
            pltpu.VMEM((2 * CH,), _i32),
            pltpu.VMEM((2 * CH,), _f32),
            pltpu.VMEM((2 * CH,), _f32),
            pltpu.VMEM((2 * CH,), _f32),
            pltpu.VMEM((2 * CH,), _f32),
            pltpu.VMEM((2 * CH,), _f32),
            pltpu.VMEM((2 * CH,), _f32),
            pltpu.VMEM((RPW,), _f32),
            pltpu.VMEM((RPW,), _f32),
            pltpu.VMEM((3 * RPW,), _f32),
            pltpu.VMEM((3 * RPW,), _f32),
            pltpu.SemaphoreType.DMA,
        ],
    )(_body)
    mask, depth, rgbf, nrmf = call(
        opacity_alpha, t, rgb[:, 0], rgb[:, 1], rgb[:, 2],
        nablas[:, 0], nablas[:, 1], nablas[:, 2], seg32, bounds)
    return (mask, depth, rgbf.reshape(NUM_RAYS, 3), nrmf.reshape(NUM_RAYS, 3))
